# baseline (speedup 1.0000x reference)
import functools

import jax
import jax.numpy as jnp
from jax import lax
from jax.experimental import pallas as pl
from jax.experimental.pallas import tpu as pltpu
from jax.experimental.pallas import tpu_sc as plsc

N = 10000
K = 16
D = 128
M = N * K
BN = 80
BM = BN * K
GRID = N // BN

_NW = 32
_L = 16
_CPW = D // _NW
_CH = 3200
_UNROLL = 8
_NCHUNK = M // _CH


def _sc_gather(table_t, idx_t):
    mesh = plsc.VectorSubcoreMesh(core_axis_name="c", subcore_axis_name="s")

    @functools.partial(
        pl.kernel,
        out_type=jax.ShapeDtypeStruct((D, M), jnp.float32),
        mesh=mesh,
        compiler_params=pltpu.CompilerParams(needs_layout_passes=False),
        scratch_types=[
            pltpu.VMEM((_CPW, N), jnp.float32),
            pltpu.VMEM((2, _CPW, _CH), jnp.int32),
            pltpu.VMEM((2, _CPW, _CH), jnp.float32),
            pltpu.SemaphoreType.DMA,
            pltpu.SemaphoreType.DMA,
        ],
    )
    def run(table_hbm, idx_hbm, out_hbm, table_v, idx_v, out_v,
            idx_sem, out_sem):
        nc = lax.axis_size("c")
        wid = lax.axis_index("s") * nc + lax.axis_index("c")
        c0 = wid * _CPW
        pltpu.sync_copy(table_hbm.at[pl.ds(c0, _CPW), :], table_v)

        def idx_dma(g, b):
            return pltpu.async_copy(
                idx_hbm.at[pl.ds(c0, _CPW), pl.ds(g * _CH, _CH)],
                idx_v.at[b], idx_sem)

        def out_dma(g, b):
            return pltpu.async_copy(
                out_v.at[b],
                out_hbm.at[pl.ds(c0, _CPW), pl.ds(g * _CH, _CH)], out_sem)

        idx_dma(0, 0)

        def chunk_body(g, _):
            b = lax.rem(g, 2)
            pltpu.make_async_copy(
                idx_hbm.at[pl.ds(c0, _CPW), pl.ds(0, _CH)],
                idx_v.at[b], idx_sem).wait()

            @pl.when(g + 1 < _NCHUNK)
            def _():
                idx_dma(g + 1, 1 - b)

            @pl.when(g >= 2)
            def _():
                pltpu.make_async_copy(
                    out_v.at[b],
                    out_hbm.at[pl.ds(c0, _CPW), pl.ds(0, _CH)],
                    out_sem).wait()

            for j in range(_CPW):
                col_j = jnp.full((_L,), j, dtype=jnp.int32)

                def inner(i, _, b=b, j=j, col_j=col_j):
                    base = i * (_L * _UNROLL)
                    for u in range(_UNROLL):
                        off = base + u * _L
                        rows = idx_v[b, j, pl.ds(off, _L)]
                        out_v[b, j, pl.ds(off, _L)] = plsc.load_gather(
                            table_v, [col_j, rows])
                    return 0

                lax.fori_loop(0, _CH // (_L * _UNROLL), inner, 0)

            out_dma(g, b)
            return 0

        lax.fori_loop(0, _NCHUNK, chunk_body, 0)
        for _ in range(2):
            pltpu.make_async_copy(
                out_v.at[0],
                out_hbm.at[pl.ds(c0, _CPW), pl.ds(0, _CH)], out_sem).wait()

    return run(table_t, idx_t)


def _silu(x):
    return x * jax.nn.sigmoid(x)


def _ln(x, g, b):
    mu = jnp.mean(x, axis=-1, keepdims=True)
    var = jnp.mean((x - mu) ** 2, axis=-1, keepdims=True)
    return (x - mu) * jax.lax.rsqrt(var + 1e-5) * g + b


def _node_block_kernel(nodes_ref, g_ref, edges_ref, mask_ref,
                       w0a_ref, w0b_ref, w0c_ref, b0_ref,
                       w1_ref, b1_ref, w2_ref, b2_ref, w3_ref, b3_ref,
                       wf0_ref, bf0_ref, wf1_ref, bf1_ref,
                       ln1g_ref, ln1b_ref, ln2g_ref, ln2b_ref,
                       out_ref):
    nodes_blk = nodes_ref[...]
    g_blk = g_ref[...]
    edges_blk = edges_ref[...]
    mask_blk = mask_ref[...]

    row_of = lax.broadcasted_iota(jnp.int32, (BM, BN), 0) // K
    col_of = lax.broadcasted_iota(jnp.int32, (BM, BN), 1)
    R = (row_of == col_of).astype(jnp.float32)

    p = jnp.dot(nodes_blk, w0a_ref[...], preferred_element_type=jnp.float32)
    h = jnp.dot(R, p, preferred_element_type=jnp.float32)
    h += lax.dot_general(g_blk, w0b_ref[...], (((0,), (0,)), ((), ())),
                         preferred_element_type=jnp.float32)
    h += jnp.dot(edges_blk, w0c_ref[...], preferred_element_type=jnp.float32)
    h = _silu(h + b0_ref[...])
    h = _silu(jnp.dot(h, w1_ref[...], preferred_element_type=jnp.float32) + b1_ref[...])
    h = _silu(jnp.dot(h, w2_ref[...], preferred_element_type=jnp.float32) + b2_ref[...])
    h = jnp.dot(h, w3_ref[...], preferred_element_type=jnp.float32) + b3_ref[...]

    hm = h * mask_blk
    nodes1 = jnp.dot(R.T, hm, preferred_element_type=jnp.float32)

    x = _ln(nodes_blk + nodes1, ln1g_ref[...], ln1b_ref[...])
    y = _silu(jnp.dot(x, wf0_ref[...], preferred_element_type=jnp.float32) + bf0_ref[...])
    y = jnp.dot(y, wf1_ref[...], preferred_element_type=jnp.float32) + bf1_ref[...]
    out_ref[...] = _ln(y + x, ln2g_ref[...], ln2b_ref[...])


def _edge_block_kernel(nodes_ref, g_ref, edges_ref,
                       w0a_ref, w0b_ref, w0c_ref, b0_ref,
                       w1_ref, b1_ref, w2_ref, b2_ref, w3_ref, b3_ref,
                       elng_ref, elnb_ref,
                       out_ref):
    nodes_blk = nodes_ref[...]
    g_blk = g_ref[...]
    edges_blk = edges_ref[...]

    row_of = lax.broadcasted_iota(jnp.int32, (BM, BN), 0) // K
    col_of = lax.broadcasted_iota(jnp.int32, (BM, BN), 1)
    R = (row_of == col_of).astype(jnp.float32)

    p = jnp.dot(nodes_blk, w0a_ref[...], preferred_element_type=jnp.float32)
    h = jnp.dot(R, p, preferred_element_type=jnp.float32)
    h += lax.dot_general(g_blk, w0b_ref[...], (((0,), (0,)), ((), ())),
                         preferred_element_type=jnp.float32)
    h += jnp.dot(edges_blk, w0c_ref[...], preferred_element_type=jnp.float32)
    h = _silu(h + b0_ref[...])
    h = _silu(jnp.dot(h, w1_ref[...], preferred_element_type=jnp.float32) + b1_ref[...])
    h = _silu(jnp.dot(h, w2_ref[...], preferred_element_type=jnp.float32) + b2_ref[...])
    h = jnp.dot(h, w3_ref[...], preferred_element_type=jnp.float32) + b3_ref[...]

    out_ref[...] = _ln(edges_blk + h, elng_ref[...], elnb_ref[...])


def _full_spec(shape):
    return pl.BlockSpec(shape, lambda i: tuple(0 for _ in shape))


def _node_pass(nodes2d, g, edges2d, mask_col,
               w0a, w0b, w0c, b0, w1, b1, w2, b2, w3, b3,
               wf0, bf0, wf1, bf1, ln1g, ln1b, ln2g, ln2b):
    in_specs = [
        pl.BlockSpec((BN, D), lambda i: (i, 0)),
        pl.BlockSpec((D, BM), lambda i: (0, i)),
        pl.BlockSpec((BM, D), lambda i: (i, 0)),
        pl.BlockSpec((BM, 1), lambda i: (i, 0)),
        _full_spec((D, D)), _full_spec((D, D)), _full_spec((D, D)),
        _full_spec((1, D)),
        _full_spec((D, D)), _full_spec((1, D)),
        _full_spec((D, D)), _full_spec((1, D)),
        _full_spec((D, D)), _full_spec((1, D)),
        _full_spec((D, 4 * D)), _full_spec((1, 4 * D)),
        _full_spec((4 * D, D)), _full_spec((1, D)),
        _full_spec((1, D)), _full_spec((1, D)),
        _full_spec((1, D)), _full_spec((1, D)),
    ]
    return pl.pallas_call(
        _node_block_kernel,
        grid=(GRID,),
        in_specs=in_specs,
        out_specs=pl.BlockSpec((BN, D), lambda i: (i, 0)),
        out_shape=jax.ShapeDtypeStruct((N, D), jnp.float32),
    )(nodes2d, g, edges2d, mask_col,
      w0a, w0b, w0c, b0, w1, b1, w2, b2, w3, b3,
      wf0, bf0, wf1, bf1, ln1g, ln1b, ln2g, ln2b)


def _edge_pass(nodes2d, g, edges2d,
               w0a, w0b, w0c, b0, w1, b1, w2, b2, w3, b3, elng, elnb):
    in_specs = [
        pl.BlockSpec((BN, D), lambda i: (i, 0)),
        pl.BlockSpec((D, BM), lambda i: (0, i)),
        pl.BlockSpec((BM, D), lambda i: (i, 0)),
        _full_spec((D, D)), _full_spec((D, D)), _full_spec((D, D)),
        _full_spec((1, D)),
        _full_spec((D, D)), _full_spec((1, D)),
        _full_spec((D, D)), _full_spec((1, D)),
        _full_spec((D, D)), _full_spec((1, D)),
        _full_spec((1, D)), _full_spec((1, D)),
    ]
    return pl.pallas_call(
        _edge_block_kernel,
        grid=(GRID,),
        in_specs=in_specs,
        out_specs=pl.BlockSpec((BM, D), lambda i: (i, 0)),
        out_shape=jax.ShapeDtypeStruct((M, D), jnp.float32),
    )(nodes2d, g, edges2d,
      w0a, w0b, w0c, b0, w1, b1, w2, b2, w3, b3, elng, elnb)


def kernel(nodes, edges, nbrs, nbr_mask,
           node_W0, node_b0, node_W1, node_b1, node_W2, node_b2, node_W3, node_b3,
           ffn_W0, ffn_b0, ffn_W1, ffn_b1,
           edge_W0, edge_b0, edge_W1, edge_b1, edge_W2, edge_b2, edge_W3, edge_b3,
           ln1_g, ln1_b, ln2_g, ln2_b, eln_g, eln_b):
    Z = nodes.shape[0]
    nodes2d = nodes.reshape(N, D)
    edges2d = edges.reshape(M, D)
    nbrs2d = nbrs.reshape(M, D)
    mask_col = nbr_mask.reshape(M, 1)

    def row(v):
        return v.reshape(1, -1)

    nbrs_t = jnp.transpose(nbrs2d)
    g1 = _sc_gather(jnp.transpose(nodes2d), nbrs_t)

    nodes2 = _node_pass(
        nodes2d, g1, edges2d, mask_col,
        node_W0[0:D], node_W0[D:2 * D], node_W0[2 * D:3 * D], row(node_b0),
        node_W1, row(node_b1), node_W2, row(node_b2), node_W3, row(node_b3),
        ffn_W0, row(ffn_b0), ffn_W1, row(ffn_b1),
        row(ln1_g), row(ln1_b), row(ln2_g), row(ln2_b))

    g2 = _sc_gather(jnp.transpose(nodes2), nbrs_t)

    edges_out = _edge_pass(
        nodes2, g2, edges2d,
        edge_W0[0:D], edge_W0[D:2 * D], edge_W0[2 * D:3 * D], row(edge_b0),
        edge_W1, row(edge_b1), edge_W2, row(edge_b2), edge_W3, row(edge_b3),
        row(eln_g), row(eln_b))

    return (nodes2.reshape(Z, N, D), edges_out.reshape(Z, N, K, D))


# speedup vs baseline: 1.6023x; 1.6023x over previous
import functools

import jax
import jax.numpy as jnp
from jax import lax
from jax.experimental import pallas as pl
from jax.experimental.pallas import tpu as pltpu
from jax.experimental.pallas import tpu_sc as plsc

N = 10000
K = 16
D = 128
M = N * K
BN = 80
BM = BN * K
GRID = N // BN

_NW = 32
_L = 16
_CPW = D // _NW
_CH = 3200
_UNROLL = 8
_NCHUNK = M // _CH


def _sc_gather(table_t, idx_t):
    mesh = plsc.VectorSubcoreMesh(core_axis_name="c", subcore_axis_name="s")

    @functools.partial(
        pl.kernel,
        out_type=jax.ShapeDtypeStruct((D, M), jnp.float32),
        mesh=mesh,
        compiler_params=pltpu.CompilerParams(needs_layout_passes=False),
        scratch_types=[
            pltpu.VMEM((_CPW, N), jnp.float32),
            pltpu.VMEM((2, _CPW, _CH), jnp.int32),
            pltpu.VMEM((2, _CPW, _CH), jnp.float32),
            pltpu.SemaphoreType.DMA,
            pltpu.SemaphoreType.DMA,
        ],
    )
    def run(table_hbm, idx_hbm, out_hbm, table_v, idx_v, out_v,
            idx_sem, out_sem):
        nc = lax.axis_size("c")
        wid = lax.axis_index("s") * nc + lax.axis_index("c")
        c0 = wid * _CPW
        pltpu.sync_copy(table_hbm.at[pl.ds(c0, _CPW), :], table_v)

        def idx_dma(g, b):
            return pltpu.async_copy(
                idx_hbm.at[pl.ds(c0, _CPW), pl.ds(g * _CH, _CH)],
                idx_v.at[b], idx_sem)

        def out_dma(g, b):
            return pltpu.async_copy(
                out_v.at[b],
                out_hbm.at[pl.ds(c0, _CPW), pl.ds(g * _CH, _CH)], out_sem)

        idx_dma(0, 0)

        def pair_body(go, _):
            for b in range(2):
                g = go * 2 + b
                pltpu.make_async_copy(
                    idx_hbm.at[pl.ds(c0, _CPW), pl.ds(0, _CH)],
                    idx_v.at[b], idx_sem).wait()

                @pl.when(g + 1 < _NCHUNK)
                def _(g=g, b=b):
                    idx_dma(g + 1, 1 - b)

                @pl.when(g >= 2)
                def _(b=b):
                    pltpu.make_async_copy(
                        out_v.at[b],
                        out_hbm.at[pl.ds(c0, _CPW), pl.ds(0, _CH)],
                        out_sem).wait()

                for j in range(_CPW):
                    col_j = jnp.full((_L,), j, dtype=jnp.int32)

                    def inner(i, _, b=b, j=j, col_j=col_j):
                        base = i * (_L * _UNROLL)
                        for u in range(_UNROLL):
                            off = base + u * _L
                            rows = idx_v[b, j, pl.ds(off, _L)]
                            out_v[b, j, pl.ds(off, _L)] = plsc.load_gather(
                                table_v, [col_j, rows])
                        return 0

                    lax.fori_loop(0, _CH // (_L * _UNROLL), inner, 0)

                out_dma(g, b)
            return 0

        lax.fori_loop(0, _NCHUNK // 2, pair_body, 0)
        for _ in range(2):
            pltpu.make_async_copy(
                out_v.at[0],
                out_hbm.at[pl.ds(c0, _CPW), pl.ds(0, _CH)], out_sem).wait()

    return run(table_t, idx_t)


def _silu(x):
    return x * jax.nn.sigmoid(x)


def _ln(x, g, b):
    mu = jnp.mean(x, axis=-1, keepdims=True)
    var = jnp.mean((x - mu) ** 2, axis=-1, keepdims=True)
    return (x - mu) * jax.lax.rsqrt(var + 1e-5) * g + b


def _node_block_kernel(nodes_ref, g_ref, edges_ref, mask_ref,
                       w0a_ref, w0b_ref, w0c_ref, b0_ref,
                       w1_ref, b1_ref, w2_ref, b2_ref, w3_ref, b3_ref,
                       wf0_ref, bf0_ref, wf1_ref, bf1_ref,
                       ln1g_ref, ln1b_ref, ln2g_ref, ln2b_ref,
                       out_ref):
    nodes_blk = nodes_ref[...]
    g_blk = g_ref[...]
    edges_blk = edges_ref[...]
    mask_blk = mask_ref[...]

    row_of = lax.broadcasted_iota(jnp.int32, (BM, BN), 0) // K
    col_of = lax.broadcasted_iota(jnp.int32, (BM, BN), 1)
    R = (row_of == col_of).astype(jnp.float32)

    p = jnp.dot(nodes_blk, w0a_ref[...], preferred_element_type=jnp.float32)
    h = jnp.dot(R, p, preferred_element_type=jnp.float32)
    h += lax.dot_general(g_blk, w0b_ref[...], (((0,), (0,)), ((), ())),
                         preferred_element_type=jnp.float32)
    h += jnp.dot(edges_blk, w0c_ref[...], preferred_element_type=jnp.float32)
    h = _silu(h + b0_ref[...])
    h = _silu(jnp.dot(h, w1_ref[...], preferred_element_type=jnp.float32) + b1_ref[...])
    h = _silu(jnp.dot(h, w2_ref[...], preferred_element_type=jnp.float32) + b2_ref[...])
    h = jnp.dot(h, w3_ref[...], preferred_element_type=jnp.float32) + b3_ref[...]

    hm = h * mask_blk
    nodes1 = jnp.dot(R.T, hm, preferred_element_type=jnp.float32)

    x = _ln(nodes_blk + nodes1, ln1g_ref[...], ln1b_ref[...])
    y = _silu(jnp.dot(x, wf0_ref[...], preferred_element_type=jnp.float32) + bf0_ref[...])
    y = jnp.dot(y, wf1_ref[...], preferred_element_type=jnp.float32) + bf1_ref[...]
    out_ref[...] = _ln(y + x, ln2g_ref[...], ln2b_ref[...])


def _edge_block_kernel(nodes_ref, g_ref, edges_ref,
                       w0a_ref, w0b_ref, w0c_ref, b0_ref,
                       w1_ref, b1_ref, w2_ref, b2_ref, w3_ref, b3_ref,
                       elng_ref, elnb_ref,
                       out_ref):
    nodes_blk = nodes_ref[...]
    g_blk = g_ref[...]
    edges_blk = edges_ref[...]

    row_of = lax.broadcasted_iota(jnp.int32, (BM, BN), 0) // K
    col_of = lax.broadcasted_iota(jnp.int32, (BM, BN), 1)
    R = (row_of == col_of).astype(jnp.float32)

    p = jnp.dot(nodes_blk, w0a_ref[...], preferred_element_type=jnp.float32)
    h = jnp.dot(R, p, preferred_element_type=jnp.float32)
    h += lax.dot_general(g_blk, w0b_ref[...], (((0,), (0,)), ((), ())),
                         preferred_element_type=jnp.float32)
    h += jnp.dot(edges_blk, w0c_ref[...], preferred_element_type=jnp.float32)
    h = _silu(h + b0_ref[...])
    h = _silu(jnp.dot(h, w1_ref[...], preferred_element_type=jnp.float32) + b1_ref[...])
    h = _silu(jnp.dot(h, w2_ref[...], preferred_element_type=jnp.float32) + b2_ref[...])
    h = jnp.dot(h, w3_ref[...], preferred_element_type=jnp.float32) + b3_ref[...]

    out_ref[...] = _ln(edges_blk + h, elng_ref[...], elnb_ref[...])


def _full_spec(shape):
    return pl.BlockSpec(shape, lambda i: tuple(0 for _ in shape))


def _node_pass(nodes2d, g, edges2d, mask_col,
               w0a, w0b, w0c, b0, w1, b1, w2, b2, w3, b3,
               wf0, bf0, wf1, bf1, ln1g, ln1b, ln2g, ln2b):
    in_specs = [
        pl.BlockSpec((BN, D), lambda i: (i, 0)),
        pl.BlockSpec((D, BM), lambda i: (0, i)),
        pl.BlockSpec((BM, D), lambda i: (i, 0)),
        pl.BlockSpec((BM, 1), lambda i: (i, 0)),
        _full_spec((D, D)), _full_spec((D, D)), _full_spec((D, D)),
        _full_spec((1, D)),
        _full_spec((D, D)), _full_spec((1, D)),
        _full_spec((D, D)), _full_spec((1, D)),
        _full_spec((D, D)), _full_spec((1, D)),
        _full_spec((D, 4 * D)), _full_spec((1, 4 * D)),
        _full_spec((4 * D, D)), _full_spec((1, D)),
        _full_spec((1, D)), _full_spec((1, D)),
        _full_spec((1, D)), _full_spec((1, D)),
    ]
    return pl.pallas_call(
        _node_block_kernel,
        grid=(GRID,),
        in_specs=in_specs,
        out_specs=pl.BlockSpec((BN, D), lambda i: (i, 0)),
        out_shape=jax.ShapeDtypeStruct((N, D), jnp.float32),
    )(nodes2d, g, edges2d, mask_col,
      w0a, w0b, w0c, b0, w1, b1, w2, b2, w3, b3,
      wf0, bf0, wf1, bf1, ln1g, ln1b, ln2g, ln2b)


def _edge_pass(nodes2d, g, edges2d,
               w0a, w0b, w0c, b0, w1, b1, w2, b2, w3, b3, elng, elnb):
    in_specs = [
        pl.BlockSpec((BN, D), lambda i: (i, 0)),
        pl.BlockSpec((D, BM), lambda i: (0, i)),
        pl.BlockSpec((BM, D), lambda i: (i, 0)),
        _full_spec((D, D)), _full_spec((D, D)), _full_spec((D, D)),
        _full_spec((1, D)),
        _full_spec((D, D)), _full_spec((1, D)),
        _full_spec((D, D)), _full_spec((1, D)),
        _full_spec((D, D)), _full_spec((1, D)),
        _full_spec((1, D)), _full_spec((1, D)),
    ]
    return pl.pallas_call(
        _edge_block_kernel,
        grid=(GRID,),
        in_specs=in_specs,
        out_specs=pl.BlockSpec((BM, D), lambda i: (i, 0)),
        out_shape=jax.ShapeDtypeStruct((M, D), jnp.float32),
    )(nodes2d, g, edges2d,
      w0a, w0b, w0c, b0, w1, b1, w2, b2, w3, b3, elng, elnb)


def kernel(nodes, edges, nbrs, nbr_mask,
           node_W0, node_b0, node_W1, node_b1, node_W2, node_b2, node_W3, node_b3,
           ffn_W0, ffn_b0, ffn_W1, ffn_b1,
           edge_W0, edge_b0, edge_W1, edge_b1, edge_W2, edge_b2, edge_W3, edge_b3,
           ln1_g, ln1_b, ln2_g, ln2_b, eln_g, eln_b):
    Z = nodes.shape[0]
    nodes2d = nodes.reshape(N, D)
    edges2d = edges.reshape(M, D)
    nbrs2d = nbrs.reshape(M, D)
    mask_col = nbr_mask.reshape(M, 1)

    def row(v):
        return v.reshape(1, -1)

    nbrs_t = jnp.transpose(nbrs2d)
    g1 = _sc_gather(jnp.transpose(nodes2d), nbrs_t)

    nodes2 = _node_pass(
        nodes2d, g1, edges2d, mask_col,
        node_W0[0:D], node_W0[D:2 * D], node_W0[2 * D:3 * D], row(node_b0),
        node_W1, row(node_b1), node_W2, row(node_b2), node_W3, row(node_b3),
        ffn_W0, row(ffn_b0), ffn_W1, row(ffn_b1),
        row(ln1_g), row(ln1_b), row(ln2_g), row(ln2_b))

    g2 = _sc_gather(jnp.transpose(nodes2), nbrs_t)

    edges_out = _edge_pass(
        nodes2, g2, edges2d,
        edge_W0[0:D], edge_W0[D:2 * D], edge_W0[2 * D:3 * D], row(edge_b0),
        edge_W1, row(edge_b1), edge_W2, row(edge_b2), edge_W3, row(edge_b3),
        row(eln_g), row(eln_b))

    return (nodes2.reshape(Z, N, D), edges_out.reshape(Z, N, K, D))
